# baseline (speedup 1.0000x reference)
import functools
import jax
import jax.numpy as jnp
from jax import lax
from jax.experimental import pallas as pl
from jax.experimental.pallas import tpu as pltpu
from jax.experimental.pallas import tpu_sc as plsc


def kernel(x, pos_table):
    B, S, D = x.shape
    info = plsc.get_sparse_core_info()
    NC, NS, L = info.num_cores, info.num_subcores, info.num_lanes
    NW = NC * NS
    rows_w = S // NW
    C = 32
    n_chunks = rows_w // C
    CW = C * D

    x_flat = x.reshape(B * S * D)
    pos_flat = pos_table[:S].reshape(S * D)
    mesh = plsc.VectorSubcoreMesh(core_axis_name="c", subcore_axis_name="s")

    @functools.partial(
        pl.kernel,
        mesh=mesh,
        out_type=jax.ShapeDtypeStruct((B * S * D,), jnp.float32),
        scratch_types=[
            pltpu.VMEM((CW,), jnp.float32),
            pltpu.VMEM((CW,), jnp.float32),
            pltpu.VMEM((CW,), jnp.float32),
            pltpu.SemaphoreType.DMA,
            pltpu.SemaphoreType.DMA,
            pltpu.SemaphoreType.DMA,
        ],
    )
    def sc_k(x_hbm, pos_hbm, out_hbm, pos_v, xa, xb, lsem, ssem, psem):
        wid = lax.axis_index("s") * NC + lax.axis_index("c")
        base = wid * rows_w * D
        xbufs = (xa, xb)

        def x_off(it):
            c, b = divmod(it, B)
            return b * S * D + base + c * CW

        n_it = n_chunks * B
        pltpu.async_copy(pos_hbm.at[pl.ds(base, CW)], pos_v, psem)
        pltpu.async_copy(x_hbm.at[pl.ds(x_off(0), CW)], xbufs[0], lsem)

        for it in range(n_it):
            buf = xbufs[it % 2]
            nbuf = xbufs[(it + 1) % 2]
            if it + 1 < n_it:
                if it + 1 >= 2:
                    pltpu.make_async_copy(nbuf, out_hbm.at[pl.ds(0, CW)], ssem).wait()
                pltpu.async_copy(
                    x_hbm.at[pl.ds(x_off(it + 1), CW)], nbuf, lsem)
            pltpu.make_async_copy(x_hbm.at[pl.ds(x_off(it), CW)], buf, lsem).wait()
            if it % B == 0:
                pltpu.make_async_copy(pos_hbm.at[pl.ds(0, CW)], pos_v, psem).wait()

            UNROLL = 16

            def body(i, _):
                for k in range(UNROLL):
                    off = i * (UNROLL * L) + k * L
                    v = pos_v[pl.ds(off, L)]
                    plsc.addupdate(buf.at[pl.ds(off, L)], v)
                return 0

            lax.fori_loop(0, CW // (UNROLL * L), body, 0)

            if it % B == B - 1 and it + 1 < n_it:
                c_next = (it + 1) // B
                pltpu.async_copy(
                    pos_hbm.at[pl.ds(base + c_next * CW, CW)], pos_v, psem)
            pltpu.async_copy(buf, out_hbm.at[pl.ds(x_off(it), CW)], ssem)

        pltpu.make_async_copy(xbufs[0], out_hbm.at[pl.ds(0, CW)], ssem).wait()
        pltpu.make_async_copy(xbufs[1], out_hbm.at[pl.ds(0, CW)], ssem).wait()

    out = sc_k(x_flat, pos_flat)
    return out.reshape(B, S, D)


# speedup vs baseline: 4.4635x; 4.4635x over previous
import jax
import jax.numpy as jnp
from jax.experimental import pallas as pl


def _add_kernel(x_ref, p_ref, o_ref):
    o_ref[...] = x_ref[...] + p_ref[...]


def kernel(x, pos_table):
    B, S, D = x.shape
    BS = 2048
    grid = (S // BS, B)
    return pl.pallas_call(
        _add_kernel,
        grid=grid,
        in_specs=[
            pl.BlockSpec((1, BS, D), lambda s, b: (b, s, 0)),
            pl.BlockSpec((BS, D), lambda s, b: (s, 0)),
        ],
        out_specs=pl.BlockSpec((1, BS, D), lambda s, b: (b, s, 0)),
        out_shape=jax.ShapeDtypeStruct((B, S, D), x.dtype),
    )(x, pos_table[:S])
